# baseline (speedup 1.0000x reference)
import functools

import jax
import jax.numpy as jnp
from jax import lax
from jax.experimental import pallas as pl
from jax.experimental.pallas import tpu as pltpu
from jax.experimental.pallas import tpu_sc as plsc

_NC = 2
_NS = 16
_L = 16


def _build_sc_call(B, N, T):
    NW = _NC * _NS
    bpw = B // NW
    nrow = bpw // 128
    nj = bpw // _L

    mesh = plsc.VectorSubcoreMesh(core_axis_name="c", subcore_axis_name="s")

    @functools.partial(
        pl.kernel,
        mesh=mesh,
        out_type=(
            jax.ShapeDtypeStruct((B,), jnp.float32),
            jax.ShapeDtypeStruct((NW, _L), jnp.float32),
        ),
        scratch_types=[
            pltpu.VMEM((bpw,), jnp.int32),
            pltpu.VMEM((bpw,), jnp.int32),
            pltpu.VMEM((nrow, 128), jnp.int32),
            pltpu.VMEM((nrow, 128), jnp.int32),
            pltpu.VMEM((bpw,), jnp.float32),
            pltpu.VMEM((bpw,), jnp.float32),
            pltpu.VMEM((T, bpw), jnp.float32),
            pltpu.VMEM((bpw,), jnp.float32),
            pltpu.VMEM((bpw,), jnp.float32),
            pltpu.VMEM((bpw,), jnp.float32),
            pltpu.VMEM((_L,), jnp.float32),
            pltpu.VMEM((_L,), jnp.float32),
            pltpu.VMEM((_L,), jnp.float32),
            pltpu.SemaphoreType.DMA,
        ],
    )
    def sc_call(qf_hbm, nqf_hbm, act_hbm, nact_hbm, rew_hbm, don_hbm, w_hbm,
                g_hbm, gT_hbm,
                td_hbm, lp_hbm,
                act_v, nact_v, qidx_v, nidx_v, qg_v, ng_v, rew_v, don_v, w_v,
                td_v, g_v, gT_v, lacc_v, sem):
        wid = lax.axis_index("s") * _NC + lax.axis_index("c")
        base = wid * bpw

        pltpu.sync_copy(act_hbm.at[pl.ds(base, bpw)], act_v)
        pltpu.sync_copy(nact_hbm.at[pl.ds(base, bpw)], nact_v)
        for t in range(T):
            pltpu.sync_copy(rew_hbm.at[t, pl.ds(base, bpw)], rew_v.at[t])
        pltpu.sync_copy(don_hbm.at[pl.ds(base, bpw)], don_v)
        pltpu.sync_copy(w_hbm.at[pl.ds(base, bpw)], w_v)
        pltpu.sync_copy(g_hbm, g_v)
        pltpu.sync_copy(gT_hbm, gT_v)

        iota = lax.iota(jnp.int32, _L)
        for j in range(nj):
            sl = pl.ds(j * _L, _L)
            rows = (base + j * _L) + iota
            qidx_v[j // 8, pl.ds((j % 8) * _L, _L)] = rows * N + act_v[sl]
            nidx_v[j // 8, pl.ds((j % 8) * _L, _L)] = rows * N + nact_v[sl]

        copies = []
        for r in range(nrow):
            copies.append(pltpu.async_copy(
                qf_hbm.at[qidx_v.at[r]], qg_v.at[pl.ds(r * 128, 128)], sem))
            copies.append(pltpu.async_copy(
                nqf_hbm.at[nidx_v.at[r]], ng_v.at[pl.ds(r * 128, 128)], sem))
        for c in copies:
            c.wait()

        g = g_v[...]
        gT = gT_v[...]
        lacc = jnp.zeros((_L,), jnp.float32)
        for j in range(nj):
            sl = pl.ds(j * _L, _L)
            acc = rew_v[T - 1, sl]
            for t in range(T - 2, -1, -1):
                acc = rew_v[t, sl] + g * acc
            nr = acc + gT * (1.0 - don_v[sl]) * ng_v[sl]
            diff = qg_v[sl] - nr
            td = diff * diff
            td_v[sl] = td
            lacc = lacc + td * w_v[sl]
        lacc_v[...] = lacc

        pltpu.sync_copy(td_v, td_hbm.at[pl.ds(base, bpw)])
        pltpu.sync_copy(lacc_v, lp_hbm.at[wid])

    return sc_call


def kernel(q, next_n_q, action, next_n_action, reward, done, weight, gamma):
    B, N = q.shape
    T = reward.shape[0]
    gamma_f = jnp.asarray(gamma, jnp.float32)
    gvec = jnp.full((_L,), gamma_f, jnp.float32)
    gTvec = jnp.full((_L,), gamma_f ** T, jnp.float32)
    sc_call = _build_sc_call(B, N, T)
    td_err, lparts = sc_call(
        q.reshape(-1), next_n_q.reshape(-1),
        action.astype(jnp.int32), next_n_action.astype(jnp.int32),
        reward, done.astype(jnp.float32), weight, gvec, gTvec)
    loss = jnp.sum(lparts) / jnp.float32(B)
    return loss, td_err


# speedup vs baseline: 1.3870x; 1.3870x over previous
import functools

import jax
import jax.numpy as jnp
from jax import lax
from jax.experimental import pallas as pl
from jax.experimental.pallas import tpu as pltpu
from jax.experimental.pallas import tpu_sc as plsc

_NC = 2
_NS = 16
_L = 16
_BR = 1024
_RT = 8192
_CH = 16


def _build_tc_call(B, N, T, RT):
    grid = RT // _BR

    def tc_body(q_ref, nq_ref, a_ref, na_ref, rt_ref, dn_ref, w_ref, fac_ref,
                td_ref, lp_ref):
        a = a_ref[...]
        na = na_ref[...]
        cols = lax.broadcasted_iota(jnp.int32, (_BR, N), 1)
        qsa = jnp.sum(jnp.where(cols == a[:, None], q_ref[...], 0.0), axis=1)
        tsa = jnp.sum(jnp.where(cols == na[:, None], nq_ref[...], 0.0), axis=1)
        fac = fac_ref[...]
        nstep = jnp.sum(rt_ref[...] * fac, axis=1)
        tcols = lax.broadcasted_iota(jnp.int32, (1, 16), 1)
        gT = jnp.sum(jnp.where(tcols == T, fac, 0.0))
        ret = nstep + gT * (1.0 - dn_ref[...]) * tsa
        diff = qsa - ret
        td = diff * diff
        td_ref[...] = td
        lp_ref[...] = jnp.full((8, 128), jnp.sum(td * w_ref[...]), jnp.float32)

    return pl.pallas_call(
        tc_body,
        grid=(grid,),
        in_specs=[
            pl.BlockSpec((_BR, N), lambda i: (i, 0)),
            pl.BlockSpec((_BR, N), lambda i: (i, 0)),
            pl.BlockSpec((_BR,), lambda i: (i,)),
            pl.BlockSpec((_BR,), lambda i: (i,)),
            pl.BlockSpec((_BR, 16), lambda i: (i, 0)),
            pl.BlockSpec((_BR,), lambda i: (i,)),
            pl.BlockSpec((_BR,), lambda i: (i,)),
            pl.BlockSpec((1, 16), lambda i: (0, 0)),
        ],
        out_specs=[
            pl.BlockSpec((_BR,), lambda i: (i,)),
            pl.BlockSpec((8, 128), lambda i: (i, 0)),
        ],
        out_shape=[
            jax.ShapeDtypeStruct((RT,), jnp.float32),
            jax.ShapeDtypeStruct((grid * 8, 128), jnp.float32),
        ],
        compiler_params=pltpu.CompilerParams(
            dimension_semantics=("arbitrary",)),
    )


def _build_sc_call(B, N, T, RT):
    NW = _NC * _NS
    RS = B - RT
    rpw = RS // NW
    nch = rpw // _CH
    nj = rpw // _L

    mesh = plsc.VectorSubcoreMesh(core_axis_name="c", subcore_axis_name="s")

    @functools.partial(
        pl.kernel,
        mesh=mesh,
        out_type=(
            jax.ShapeDtypeStruct((RS,), jnp.float32),
            jax.ShapeDtypeStruct((NW * 128,), jnp.float32),
        ),
        compiler_params=pltpu.CompilerParams(use_tc_tiling_on_sc=True,
                                             needs_layout_passes=False),
        scratch_types=[
            pltpu.VMEM((rpw,), jnp.int32),
            pltpu.VMEM((rpw,), jnp.int32),
            pltpu.VMEM((_CH, N), jnp.float32),
            pltpu.VMEM((_CH, N), jnp.float32),
            pltpu.VMEM((_CH, N), jnp.float32),
            pltpu.VMEM((_CH, N), jnp.float32),
            pltpu.VMEM((T * rpw,), jnp.float32),
            pltpu.VMEM((rpw,), jnp.float32),
            pltpu.VMEM((rpw,), jnp.float32),
            pltpu.VMEM((rpw,), jnp.float32),
            pltpu.VMEM((128,), jnp.float32),
            pltpu.VMEM((128,), jnp.float32),
            pltpu.VMEM((128,), jnp.float32),
            pltpu.SemaphoreType.DMA,
        ],
    )
    def sc_call(q_hbm, nq_hbm, act_hbm, nact_hbm, rew_hbm, don_hbm, w_hbm,
                g_hbm, gT_hbm,
                td_hbm, lp_hbm,
                act_v, nact_v, qb0, qb1, nb0, nb1, rew_v, don_v, w_v,
                td_v, g_v, gT_v, lacc_v, sem):
        wid = lax.axis_index("s") * _NC + lax.axis_index("c")
        base = RT + wid * rpw
        obase = wid * rpw

        pltpu.sync_copy(act_hbm.at[pl.ds(base, rpw)], act_v)
        pltpu.sync_copy(nact_hbm.at[pl.ds(base, rpw)], nact_v)
        for t in range(T):
            pltpu.sync_copy(rew_hbm.at[pl.ds(t * B + base, rpw)],
                            rew_v.at[pl.ds(t * rpw, rpw)])
        pltpu.sync_copy(don_hbm.at[pl.ds(base, rpw)], don_v)
        pltpu.sync_copy(w_hbm.at[pl.ds(base, rpw)], w_v)
        pltpu.sync_copy(g_hbm, g_v)
        pltpu.sync_copy(gT_hbm, gT_v)

        qbufs = (qb0, qb1)
        nbufs = (nb0, nb1)

        def fire(ch):
            b = ch & 1
            return (
                pltpu.async_copy(
                    q_hbm.at[pl.ds(base + ch * _CH, _CH), :], qbufs[b], sem),
                pltpu.async_copy(
                    nq_hbm.at[pl.ds(base + ch * _CH, _CH), :], nbufs[b], sem),
            )

        g = g_v[pl.ds(0, _L)]
        gT = gT_v[pl.ds(0, _L)]
        iota = lax.iota(jnp.int32, _L)
        lacc = jnp.zeros((_L,), jnp.float32)
        pend = fire(0)
        for ch in range(nch):
            cur = pend
            if ch + 1 < nch:
                pend = fire(ch + 1)
            cur[0].wait()
            cur[1].wait()
            b = ch & 1
            sl = pl.ds(ch * _CH, _L)
            qv = plsc.load_gather(qbufs[b], [iota, act_v[sl]])
            nv = plsc.load_gather(nbufs[b], [iota, nact_v[sl]])
            acc = rew_v[pl.ds((T - 1) * rpw + ch * _CH, _L)]
            for t in range(T - 2, -1, -1):
                acc = rew_v[pl.ds(t * rpw + ch * _CH, _L)] + g * acc
            nr = acc + gT * (1.0 - don_v[sl]) * nv
            diff = qv - nr
            td = diff * diff
            td_v[sl] = td
            lacc = lacc + td * w_v[sl]
        zeros = jnp.zeros((_L,), jnp.float32)
        for k in range(128 // _L):
            lacc_v[pl.ds(k * _L, _L)] = lacc if k == 0 else zeros

        pltpu.sync_copy(td_v, td_hbm.at[pl.ds(obase, rpw)])
        pltpu.sync_copy(lacc_v, lp_hbm.at[pl.ds(wid * 128, 128)])

    return sc_call


def kernel(q, next_n_q, action, next_n_action, reward, done, weight, gamma):
    B, N = q.shape
    T = reward.shape[0]
    gamma_f = jnp.asarray(gamma, jnp.float32)
    tpow = jnp.arange(16, dtype=jnp.float32)
    fac = jnp.where(tpow <= T, gamma_f ** tpow, 0.0).reshape(1, 16)
    gvec = jnp.full((128,), gamma_f, jnp.float32)
    gTvec = jnp.full((128,), gamma_f ** T, jnp.float32)
    act = action.astype(jnp.int32)
    nact = next_n_action.astype(jnp.int32)
    done_f = done.astype(jnp.float32)
    rewT = jnp.pad(reward.T, ((0, 0), (0, 16 - T)))

    tc_call = _build_tc_call(B, N, T, _RT)
    td_tc, lp_tc = tc_call(q, next_n_q, act, nact, rewT, done_f, weight, fac)

    sc_call = _build_sc_call(B, N, T, _RT)
    td_sc, lp_sc = sc_call(q, next_n_q, act, nact, reward.reshape(-1),
                           done_f, weight, gvec, gTvec)

    td_err = jnp.concatenate([td_tc, td_sc])
    loss = (jnp.sum(lp_tc) / jnp.float32(8 * 128) + jnp.sum(lp_sc)) \
        / jnp.float32(B)
    return loss, td_err
